# baseline (speedup 1.0000x reference)
import functools

import jax
import jax.numpy as jnp
from jax import lax
from jax.experimental import pallas as pl
from jax.experimental.pallas import tpu as pltpu
from jax.experimental.pallas import tpu_sc as plsc

VOCAB = 1000000
EMB = 64
BATCH = 4096
HIST = 200

TBLK = 16384
NBLK_A = 16
S4 = NBLK_A * TBLK
NBLK_Q3_LAST = 13
ROWS_RM = 4 * S4
PK = EMB // 2

NUM_CORES = 2
NUM_SUBCORES = 16
NUM_WORKERS = NUM_CORES * NUM_SUBCORES
B_PER_W = BATCH // NUM_WORKERS
C0 = 128
C1 = HIST - C0

_LANES = 16
_NV = EMB // _LANES

def _pack_body(t_ref, out_ref):
    q = pl.program_id(1)
    y = t_ref[...]
    lo = jax.lax.bitcast_convert_type(
        y[:PK, :].astype(jnp.bfloat16), jnp.uint16).astype(jnp.uint32)
    hi = jax.lax.bitcast_convert_type(
        y[PK:, :].astype(jnp.bfloat16), jnp.uint16).astype(jnp.uint32)
    word = lo | (hi << 16)
    wt = jax.lax.bitcast_convert_type(word.T, jnp.float32)
    for qq in range(4):
        @pl.when(q == qq)
        def _():
            out_ref[:, qq * PK:(qq + 1) * PK] = wt


def _pack_tc(tab_t):
    def in_imap(c, q):
        return (0, jnp.where(q < 3, q * NBLK_A + c,
                             3 * NBLK_A + jnp.minimum(c, NBLK_Q3_LAST)))

    return pl.pallas_call(
        _pack_body,
        grid=(NBLK_A, 4),
        in_specs=[pl.BlockSpec((EMB, TBLK), in_imap)],
        out_specs=pl.BlockSpec((TBLK, 128), lambda c, q: (c, 0)),
        out_shape=jax.ShapeDtypeStruct((S4, 128), jnp.float32),
    )(tab_t)


def _accum_chunk(rows_ref, n, acc):
    mask = jnp.full((_LANES,), 0xFFFF0000, jnp.uint32)
    shift = jnp.full((_LANES,), 16, jnp.uint32)

    def split(words):
        u = lax.bitcast_convert_type(words, jnp.uint32)
        e_lo = lax.bitcast_convert_type(u << shift, jnp.float32)
        e_hi = lax.bitcast_convert_type(u & mask, jnp.float32)
        return e_lo, e_hi

    def body(j, acc):
        a0, a1, a2, a3 = acc
        lo = rows_ref[j, pl.ds(0, _LANES)]
        hi = rows_ref[j, pl.ds(_LANES, _LANES)]
        e0, e2 = split(lo)
        e1, e3 = split(hi)
        return (a0 + e0, a1 + e1, a2 + e2, a3 + e3)

    return lax.fori_loop(0, n, body, acc)


def _bow_body(idx_hbm, table_hbm, out_hbm,
              idx_v, ra0, rb0, ra1, rb1, out_v, sem0, sem1):
    wid = lax.axis_index("s") * NUM_CORES + lax.axis_index("c")
    base = pl.multiple_of(wid * B_PER_W, 8)

    pltpu.sync_copy(idx_hbm.at[pl.ds(base * HIST, B_PER_W * HIST)], idx_v)

    def row_copies(b, ra, rb, sem):
        off = pl.multiple_of(b * HIST, 8)
        cp_a = pltpu.make_async_copy(
            table_hbm.at[idx_v.at[pl.ds(off, C0)]], ra, sem)
        cp_b = pltpu.make_async_copy(
            table_hbm.at[idx_v.at[pl.ds(off + C0, C1)]], rb, sem)
        return cp_a, cp_b

    def start_row(b, ra, rb, sem):
        cp_a, cp_b = row_copies(b, ra, rb, sem)
        cp_a.start()
        cp_b.start()

    def finish_row(b, ra, rb, sem):
        cp_a, cp_b = row_copies(b, ra, rb, sem)
        cp_a.wait()
        cp_b.wait()
        acc = tuple(jnp.zeros((_LANES,), jnp.float32) for _ in range(_NV))
        acc = _accum_chunk(ra, C0, acc)
        acc = _accum_chunk(rb, C1, acc)
        for d in range(_NV):
            out_v[b, pl.ds(d * _LANES, _LANES)] = acc[d]

    start_row(0, ra0, rb0, sem0)

    def outer(i, carry):
        b = i * 2
        start_row(b + 1, ra1, rb1, sem1)
        finish_row(b, ra0, rb0, sem0)

        @pl.when(b + 2 < B_PER_W)
        def _():
            start_row(b + 2, ra0, rb0, sem0)

        finish_row(b + 1, ra1, rb1, sem1)
        return carry

    lax.fori_loop(0, B_PER_W // 2, outer, 0)

    pltpu.sync_copy(out_v, out_hbm.at[pl.ds(base, B_PER_W)])


@jax.jit
def _bow(idx_flat, table_rm):
    mesh = plsc.VectorSubcoreMesh(core_axis_name="c", subcore_axis_name="s")
    run = functools.partial(
        pl.kernel,
        mesh=mesh,
        compiler_params=pltpu.CompilerParams(use_tc_tiling_on_sc=False),
        out_type=jax.ShapeDtypeStruct((BATCH, EMB), jnp.float32),
        scratch_types=[
            pltpu.VMEM((B_PER_W * HIST,), jnp.int32),
            pltpu.VMEM((C0, PK), jnp.float32),
            pltpu.VMEM((C1, PK), jnp.float32),
            pltpu.VMEM((C0, PK), jnp.float32),
            pltpu.VMEM((C1, PK), jnp.float32),
            pltpu.VMEM((B_PER_W, EMB), jnp.float32),
            pltpu.SemaphoreType.DMA,
            pltpu.SemaphoreType.DMA,
        ],
    )(_bow_body)
    return run(idx_flat, table_rm)


def kernel(indices, table):
    idx32 = indices.astype(jnp.int32)
    ridx = 4 * (idx32 & (S4 - 1)) + (idx32 >> 18)
    packed = _pack_tc(table.T)
    table_rm = packed.reshape(ROWS_RM, PK)
    return _bow(ridx.reshape(-1), table_rm)


# speedup vs baseline: 1.1867x; 1.1867x over previous
import functools

import jax
import jax.numpy as jnp
from jax import lax
from jax.experimental import pallas as pl
from jax.experimental.pallas import tpu as pltpu
from jax.experimental.pallas import tpu_sc as plsc

VOCAB = 1000000
EMB = 64
BATCH = 4096
HIST = 200

TBLK = 16384
NBLK_A = 31
SPLIT = NBLK_A * TBLK
NBLK_B_LAST = 30
VOCAB_RM = 2 * SPLIT

NUM_CORES = 2
NUM_SUBCORES = 16
NUM_WORKERS = NUM_CORES * NUM_SUBCORES
B_PER_W = BATCH // NUM_WORKERS
C0 = 128
C1 = HIST - C0

_LANES = 16
_NV = EMB // _LANES


def _pack_body(ta_ref, tb_ref, out_ref):
    a = ta_ref[...].T
    b = tb_ref[...].T
    out_ref[...] = jnp.concatenate([a, b], axis=1)


def _pack_tc(tab_t):
    return pl.pallas_call(
        _pack_body,
        grid=(NBLK_A,),
        in_specs=[
            pl.BlockSpec((EMB, TBLK), lambda c: (0, c)),
            pl.BlockSpec((EMB, TBLK),
                         lambda c: (0, NBLK_A + jnp.minimum(c, NBLK_B_LAST))),
        ],
        out_specs=pl.BlockSpec((TBLK, 128), lambda c: (c, 0)),
        out_shape=jax.ShapeDtypeStruct((SPLIT, 128), jnp.float32),
    )(tab_t, tab_t)


def _accum_chunk(rows_ref, n, acc):
    def body(j, acc):
        return tuple(
            acc[d] + rows_ref[j, pl.ds(d * _LANES, _LANES)] for d in range(_NV)
        )

    return lax.fori_loop(0, n, body, acc)


def _bow_body(idx_hbm, table_hbm, out_hbm,
              idx_v, ra0, rb0, ra1, rb1, out_v, sem0, sem1):
    wid = lax.axis_index("s") * NUM_CORES + lax.axis_index("c")
    base = pl.multiple_of(wid * B_PER_W, 8)

    pltpu.sync_copy(idx_hbm.at[pl.ds(base * HIST, B_PER_W * HIST)], idx_v)

    def row_copies(b, ra, rb, sem):
        off = pl.multiple_of(b * HIST, 8)
        cp_a = pltpu.make_async_copy(
            table_hbm.at[idx_v.at[pl.ds(off, C0)]], ra, sem)
        cp_b = pltpu.make_async_copy(
            table_hbm.at[idx_v.at[pl.ds(off + C0, C1)]], rb, sem)
        return cp_a, cp_b

    def start_row(b, ra, rb, sem):
        cp_a, cp_b = row_copies(b, ra, rb, sem)
        cp_a.start()
        cp_b.start()

    def finish_row(b, ra, rb, sem):
        cp_a, cp_b = row_copies(b, ra, rb, sem)
        cp_a.wait()
        cp_b.wait()
        acc = tuple(jnp.zeros((_LANES,), jnp.float32) for _ in range(_NV))
        acc = _accum_chunk(ra, C0, acc)
        acc = _accum_chunk(rb, C1, acc)
        for d in range(_NV):
            out_v[b, pl.ds(d * _LANES, _LANES)] = acc[d]

    start_row(0, ra0, rb0, sem0)

    def outer(i, carry):
        b = i * 2
        start_row(b + 1, ra1, rb1, sem1)
        finish_row(b, ra0, rb0, sem0)

        @pl.when(b + 2 < B_PER_W)
        def _():
            start_row(b + 2, ra0, rb0, sem0)

        finish_row(b + 1, ra1, rb1, sem1)
        return carry

    lax.fori_loop(0, B_PER_W // 2, outer, 0)

    pltpu.sync_copy(out_v, out_hbm.at[pl.ds(base, B_PER_W)])


@jax.jit
def _bow(idx_flat, table_rm):
    mesh = plsc.VectorSubcoreMesh(core_axis_name="c", subcore_axis_name="s")
    run = functools.partial(
        pl.kernel,
        mesh=mesh,
        compiler_params=pltpu.CompilerParams(use_tc_tiling_on_sc=False),
        out_type=jax.ShapeDtypeStruct((BATCH, EMB), jnp.float32),
        scratch_types=[
            pltpu.VMEM((B_PER_W * HIST,), jnp.int32),
            pltpu.VMEM((C0, EMB), jnp.float32),
            pltpu.VMEM((C1, EMB), jnp.float32),
            pltpu.VMEM((C0, EMB), jnp.float32),
            pltpu.VMEM((C1, EMB), jnp.float32),
            pltpu.VMEM((B_PER_W, EMB), jnp.float32),
            pltpu.SemaphoreType.DMA,
            pltpu.SemaphoreType.DMA,
        ],
    )(_bow_body)
    return run(idx_flat, table_rm)


def kernel(indices, table):
    idx32 = indices.astype(jnp.int32)
    ridx = jnp.where(idx32 < SPLIT, 2 * idx32, 2 * (idx32 - SPLIT) + 1)
    packed = _pack_tc(table.T)
    table_rm = packed.reshape(VOCAB_RM, EMB)
    return _bow(ridx.reshape(-1), table_rm)


# speedup vs baseline: 1.1875x; 1.0007x over previous
import functools

import jax
import jax.numpy as jnp
from jax import lax
from jax.experimental import pallas as pl
from jax.experimental.pallas import tpu as pltpu
from jax.experimental.pallas import tpu_sc as plsc

VOCAB = 1000000
EMB = 64
BATCH = 4096
HIST = 200

TBLK = 16384
NBLK_A = 31
SPLIT = NBLK_A * TBLK
NBLK_B_LAST = 30
VOCAB_RM = 2 * SPLIT

NUM_CORES = 2
NUM_SUBCORES = 16
NUM_WORKERS = NUM_CORES * NUM_SUBCORES
B_PER_W = BATCH // NUM_WORKERS
C0 = 104
C1 = HIST - C0

_LANES = 16
_NV = EMB // _LANES


def _pack_body(ta_ref, tb_ref, out_ref):
    a = ta_ref[...].T
    b = tb_ref[...].T
    out_ref[...] = jnp.concatenate([a, b], axis=1)


def _pack_tc(tab_t):
    return pl.pallas_call(
        _pack_body,
        grid=(NBLK_A,),
        in_specs=[
            pl.BlockSpec((EMB, TBLK), lambda c: (0, c)),
            pl.BlockSpec((EMB, TBLK),
                         lambda c: (0, NBLK_A + jnp.minimum(c, NBLK_B_LAST))),
        ],
        out_specs=pl.BlockSpec((TBLK, 128), lambda c: (c, 0)),
        out_shape=jax.ShapeDtypeStruct((SPLIT, 128), jnp.float32),
    )(tab_t, tab_t)


def _accum_chunk(rows_ref, n, acc):
    def body(j, acc):
        return tuple(
            acc[d] + rows_ref[j, pl.ds(d * _LANES, _LANES)] for d in range(_NV)
        )

    return lax.fori_loop(0, n, body, acc)


def _bow_body(idx_hbm, table_hbm, out_hbm,
              idx_v, ra0, rb0, ra1, rb1, out_v, sem0, sem1):
    wid = lax.axis_index("s") * NUM_CORES + lax.axis_index("c")
    base = pl.multiple_of(wid * B_PER_W, 8)

    pltpu.sync_copy(idx_hbm.at[pl.ds(base * HIST, B_PER_W * HIST)], idx_v)

    def row_copies(b, ra, rb, sem):
        off = pl.multiple_of(b * HIST, 8)
        cp_a = pltpu.make_async_copy(
            table_hbm.at[idx_v.at[pl.ds(off, C0)]], ra, sem)
        cp_b = pltpu.make_async_copy(
            table_hbm.at[idx_v.at[pl.ds(off + C0, C1)]], rb, sem)
        return cp_a, cp_b

    def start_row(b, ra, rb, sem):
        cp_a, cp_b = row_copies(b, ra, rb, sem)
        cp_a.start()
        cp_b.start()

    def finish_row(b, ra, rb, sem):
        cp_a, cp_b = row_copies(b, ra, rb, sem)
        cp_a.wait()
        cp_b.wait()
        acc = tuple(jnp.zeros((_LANES,), jnp.float32) for _ in range(_NV))
        acc = _accum_chunk(ra, C0, acc)
        acc = _accum_chunk(rb, C1, acc)
        for d in range(_NV):
            out_v[b, pl.ds(d * _LANES, _LANES)] = acc[d]

    start_row(0, ra0, rb0, sem0)

    def outer(i, carry):
        b = i * 2
        start_row(b + 1, ra1, rb1, sem1)
        finish_row(b, ra0, rb0, sem0)

        @pl.when(b + 2 < B_PER_W)
        def _():
            start_row(b + 2, ra0, rb0, sem0)

        finish_row(b + 1, ra1, rb1, sem1)
        return carry

    lax.fori_loop(0, B_PER_W // 2, outer, 0)

    pltpu.sync_copy(out_v, out_hbm.at[pl.ds(base, B_PER_W)])


@jax.jit
def _bow(idx_flat, table_rm):
    mesh = plsc.VectorSubcoreMesh(core_axis_name="c", subcore_axis_name="s")
    run = functools.partial(
        pl.kernel,
        mesh=mesh,
        compiler_params=pltpu.CompilerParams(use_tc_tiling_on_sc=False),
        out_type=jax.ShapeDtypeStruct((BATCH, EMB), jnp.float32),
        scratch_types=[
            pltpu.VMEM((B_PER_W * HIST,), jnp.int32),
            pltpu.VMEM((C0, EMB), jnp.float32),
            pltpu.VMEM((C1, EMB), jnp.float32),
            pltpu.VMEM((C0, EMB), jnp.float32),
            pltpu.VMEM((C1, EMB), jnp.float32),
            pltpu.VMEM((B_PER_W, EMB), jnp.float32),
            pltpu.SemaphoreType.DMA,
            pltpu.SemaphoreType.DMA,
        ],
    )(_bow_body)
    return run(idx_flat, table_rm)


def kernel(indices, table):
    idx32 = indices.astype(jnp.int32)
    ridx = jnp.where(idx32 < SPLIT, 2 * idx32, 2 * (idx32 - SPLIT) + 1)
    packed = _pack_tc(table.T)
    table_rm = packed.reshape(VOCAB_RM, EMB)
    return _bow(ridx.reshape(-1), table_rm)
